# baseline (speedup 1.0000x reference)
import jax
import jax.numpy as jnp
from jax import lax
from jax.experimental import pallas as pl
from jax.experimental.pallas import tpu as pltpu
from jax.experimental.pallas import tpu_sc as plsc

N_STIMULI = 20000
N_DIM = 512
N_TRIAL = 16384
R = 8
S = 9
NW = 32
TPW = N_TRIAL // NW
C = 8
NCHUNK = TPW // C
ND2 = N_DIM // 2


def _rsqrt(x):
    i = plsc.bitcast(x, jnp.int32)
    i = jnp.int32(0x5F3759DF) - (i >> 1)
    y = plsc.bitcast(i, jnp.float32)
    for _ in range(3):
        y = y * (jnp.float32(1.5) - jnp.float32(0.5) * x * y * y)
    return y


def _sc_body(ss_hbm, gid_hbm, cfg_hbm, ipt_hbm, z_hbm, attn_hbm, out_hbm,
             ss_v, gid_v, cfg_v, ip_v, attn_v, rows_v, rows1_v, d2_v, out_v,
             tmp_v, sem, sem1):
    wid = lax.axis_index("s") * 2 + lax.axis_index("c")
    tbase = wid * TPW

    pltpu.sync_copy(ss_hbm.at[pl.ds(tbase * S, TPW * S)], ss_v)
    pltpu.sync_copy(gid_hbm.at[pl.ds(tbase, TPW)], gid_v)
    pltpu.sync_copy(cfg_hbm.at[pl.ds(tbase, TPW)], cfg_v)
    for r in range(R):
        pltpu.sync_copy(
            ipt_hbm.at[pl.ds((r + 1) * N_TRIAL + tbase, TPW)], ip_v.at[r]
        )
    pltpu.sync_copy(attn_hbm, attn_v)

    lanes = lax.iota(jnp.int32, 16)
    tl = lanes % 8
    half = lanes // 8
    rot8 = (lanes + 8) % 16
    row_q = tl * S
    zero16 = jnp.zeros((16,), jnp.float32)

    row_r = [row_q + (1 + r) for r in range(R)]
    rsplat = [jnp.full((16,), r, jnp.int32) for r in range(R)]

    def start_gather(c, buf, dsem):
        pltpu.async_copy(z_hbm.at[ss_v.at[pl.ds(c * (C * S), C * S)]],
                         buf, dsem)

    def wait_gather(buf, dsem):
        pltpu.make_async_copy(
            z_hbm.at[ss_v.at[pl.ds(0, C * S)]], buf, dsem
        ).wait()

    def compute_chunk(c, buf):
        cbase = jnp.full((16,), c * C, jnp.int32)
        gid8 = plsc.load_gather(gid_v, [cbase + tl])

        @plsc.parallel_loop(0, ND2, 1, unroll=4, carry=(zero16,) * R)
        def dim_step(i, accs_c):
            col = half + jnp.full((16,), 2 * i, jnp.int32)
            q = plsc.load_gather(buf, [row_q, col])
            a = plsc.load_gather(attn_v, [gid8, col])
            new = []
            for r in range(R):
                v = plsc.load_gather(buf, [row_r[r], col])
                dlt = v - q
                new.append(accs_c[r] + a * dlt * dlt)
            return tuple(new)

        accs = dim_step
        tcol = cbase + tl
        for r in range(R):
            tmp_v[:] = accs[r]
            d2 = accs[r] + plsc.load_gather(tmp_v, [rot8])
            plsc.store_scatter(d2_v, [rsplat[r], tcol], d2)

    start_gather(0, rows_v, sem)

    def chunk_pair(j, carry):
        c0 = 2 * j
        wait_gather(rows_v, sem)
        start_gather(c0 + 1, rows1_v, sem1)
        compute_chunk(c0, rows_v)
        wait_gather(rows1_v, sem1)

        @pl.when(c0 + 2 < NCHUNK)
        def _():
            start_gather(c0 + 2, rows_v, sem)

        compute_chunk(c0 + 1, rows1_v)
        return carry

    lax.fori_loop(0, NCHUNK // 2, chunk_pair, 0)

    def group(g, carry):
        sl = pl.ds(g * 16, 16)
        sims = []
        for r in range(R):
            d2 = d2_v[r, sl]
            d = jnp.where(d2 > 0.0, d2 * _rsqrt(d2), jnp.float32(0.0))
            sims.append(jnp.exp(-d) * ip_v[r, sl])
        denom_b = sims[1]
        for r in range(2, R):
            denom_b = denom_b + sims[r]
        denom_a = denom_b + sims[0]
        p1 = sims[0] / denom_a
        p2 = (sims[1] / denom_b) * p1
        out_v[sl] = jnp.where(cfg_v[sl] == 0, p1, p2)
        return carry

    lax.fori_loop(0, TPW // 16, group, 0)
    pltpu.sync_copy(out_v, out_hbm.at[pl.ds(tbase, TPW)])


def kernel(stimulus_set, config_idx, group_id, is_present, z, attn_w):
    ss_flat = stimulus_set.reshape(-1).astype(jnp.int32)
    ipt = is_present.T.reshape(-1)
    mesh = plsc.VectorSubcoreMesh(core_axis_name="c", subcore_axis_name="s")
    fn = pl.kernel(
        _sc_body,
        out_type=jax.ShapeDtypeStruct((N_TRIAL,), jnp.float32),
        mesh=mesh,
        scratch_types=[
            pltpu.VMEM((TPW * S,), jnp.int32),
            pltpu.VMEM((TPW,), jnp.int32),
            pltpu.VMEM((TPW,), jnp.int32),
            pltpu.VMEM((R, TPW), jnp.float32),
            pltpu.VMEM((2, N_DIM), jnp.float32),
            pltpu.VMEM((C * S, N_DIM), jnp.float32),
            pltpu.VMEM((C * S, N_DIM), jnp.float32),
            pltpu.VMEM((R, TPW), jnp.float32),
            pltpu.VMEM((TPW,), jnp.float32),
            pltpu.VMEM((16,), jnp.float32),
            pltpu.SemaphoreType.DMA,
            pltpu.SemaphoreType.DMA,
        ],
        compiler_params=pltpu.CompilerParams(needs_layout_passes=False),
    )
    return fn(ss_flat, group_id.astype(jnp.int32), config_idx.astype(jnp.int32),
              ipt.astype(jnp.float32), z, attn_w)


# speedup vs baseline: 4.3202x; 4.3202x over previous
import jax
import jax.numpy as jnp
from jax import lax
from jax.experimental import pallas as pl
from jax.experimental.pallas import tpu as pltpu
from jax.experimental.pallas import tpu_sc as plsc

N_STIMULI = 20000
N_DIM = 512
N_TRIAL = 16384
R = 8
S = 9
NW = 32
TPW = N_TRIAL // NW
C = 8
NCHUNK = TPW // C
KCH = N_DIM // 16


def _rsqrt(x):
    i = plsc.bitcast(x, jnp.int32)
    i = jnp.int32(0x5F3759DF) - (i >> 1)
    y = plsc.bitcast(i, jnp.float32)
    for _ in range(3):
        y = y * (jnp.float32(1.5) - jnp.float32(0.5) * x * y * y)
    return y


def _sc_body(ss_hbm, gid_hbm, cfg_hbm, ipt_hbm, z_hbm, attn_hbm, out_hbm,
             ss_v, gid_v, cfg_v, ip_v, attn_v, rows_v, rows1_v, d2_v, out_v,
             tmp_v, sem, semb, sem1, sem1b):
    wid = lax.axis_index("s") * 2 + lax.axis_index("c")
    tbase = wid * TPW

    pltpu.sync_copy(ss_hbm.at[pl.ds(tbase * S, TPW * S)], ss_v)
    pltpu.sync_copy(gid_hbm.at[pl.ds(tbase, TPW)], gid_v)
    pltpu.sync_copy(cfg_hbm.at[pl.ds(tbase, TPW)], cfg_v)
    for r in range(R):
        pltpu.sync_copy(
            ipt_hbm.at[pl.ds((r + 1) * N_TRIAL + tbase, TPW)], ip_v.at[r]
        )
    pltpu.sync_copy(attn_hbm, attn_v)

    lanes = lax.iota(jnp.int32, 16)
    lane_lt8 = lanes < 8
    rmasks = [lanes == r for r in range(R)]
    zero16 = jnp.zeros((16,), jnp.float32)

    H1, H2 = 40, 32

    def start_gather(c, buf, dsem, dsem2):
        pltpu.async_copy(z_hbm.at[ss_v.at[pl.ds(c * (C * S), H1)]],
                         buf.at[pl.ds(0, H1)], dsem)
        pltpu.async_copy(z_hbm.at[ss_v.at[pl.ds(c * (C * S) + H1, H2)]],
                         buf.at[pl.ds(H1, H2)], dsem2)

    def wait_gather(buf, dsem, dsem2):
        pltpu.make_async_copy(z_hbm.at[ss_v.at[pl.ds(0, H1)]],
                              buf.at[pl.ds(0, H1)], dsem).wait()
        pltpu.make_async_copy(z_hbm.at[ss_v.at[pl.ds(0, H2)]],
                              buf.at[pl.ds(H1, H2)], dsem2).wait()

    def compute_chunk(c, buf):
        def trial(t, carry2):
            tglob = c * C + t
            pred = plsc.load_gather(gid_v, [jnp.full((16,), tglob,
                                                     jnp.int32)]) == 0
            qrow = t * S

            @plsc.parallel_loop(0, KCH, 1, unroll=2, carry=(zero16,) * R)
            def kstep(k, accs_c):
                sl = pl.ds(k * 16, 16)
                q = buf[qrow, sl]
                a = jnp.where(pred, attn_v[0, sl], attn_v[1, sl])
                new = []
                for r in range(R):
                    dlt = buf[qrow + 1 + r, sl] - q
                    new.append(accs_c[r] + a * dlt * dlt)
                return tuple(new)

            d2vec = zero16
            for r in range(R):
                d2vec = jnp.where(rmasks[r], jnp.full((16,),
                                                      jnp.sum(kstep[r])),
                                  d2vec)
            plsc.store_scatter(d2_v, [lanes, jnp.full((16,), tglob,
                                                      jnp.int32)],
                               d2vec, mask=lane_lt8)
            return carry2

        lax.fori_loop(0, C, trial, 0)

    start_gather(0, rows_v, sem, semb)

    def chunk_pair(j, carry):
        c0 = 2 * j
        wait_gather(rows_v, sem, semb)
        start_gather(c0 + 1, rows1_v, sem1, sem1b)
        compute_chunk(c0, rows_v)
        wait_gather(rows1_v, sem1, sem1b)

        @pl.when(c0 + 2 < NCHUNK)
        def _():
            start_gather(c0 + 2, rows_v, sem, semb)

        compute_chunk(c0 + 1, rows1_v)
        return carry

    lax.fori_loop(0, NCHUNK // 2, chunk_pair, 0)

    def group(g, carry):
        sl = pl.ds(g * 16, 16)
        sims = []
        for r in range(R):
            d2 = d2_v[r, sl]
            d = jnp.where(d2 > 0.0, d2 * _rsqrt(d2), jnp.float32(0.0))
            sims.append(jnp.exp(-d) * ip_v[r, sl])
        denom_b = sims[1]
        for r in range(2, R):
            denom_b = denom_b + sims[r]
        denom_a = denom_b + sims[0]
        p1 = sims[0] / denom_a
        p2 = (sims[1] / denom_b) * p1
        out_v[sl] = jnp.where(cfg_v[sl] == 0, p1, p2)
        return carry

    lax.fori_loop(0, TPW // 16, group, 0)
    pltpu.sync_copy(out_v, out_hbm.at[pl.ds(tbase, TPW)])


def kernel(stimulus_set, config_idx, group_id, is_present, z, attn_w):
    ss_flat = stimulus_set.reshape(-1).astype(jnp.int32)
    ipt = is_present.T.reshape(-1)
    mesh = plsc.VectorSubcoreMesh(core_axis_name="c", subcore_axis_name="s")
    fn = pl.kernel(
        _sc_body,
        out_type=jax.ShapeDtypeStruct((N_TRIAL,), jnp.float32),
        mesh=mesh,
        scratch_types=[
            pltpu.VMEM((TPW * S,), jnp.int32),
            pltpu.VMEM((TPW,), jnp.int32),
            pltpu.VMEM((TPW,), jnp.int32),
            pltpu.VMEM((R, TPW), jnp.float32),
            pltpu.VMEM((2, N_DIM), jnp.float32),
            pltpu.VMEM((C * S, N_DIM), jnp.float32),
            pltpu.VMEM((C * S, N_DIM), jnp.float32),
            pltpu.VMEM((R, TPW), jnp.float32),
            pltpu.VMEM((TPW,), jnp.float32),
            pltpu.VMEM((16,), jnp.float32),
            pltpu.SemaphoreType.DMA,
            pltpu.SemaphoreType.DMA,
            pltpu.SemaphoreType.DMA,
            pltpu.SemaphoreType.DMA,
        ],
        compiler_params=pltpu.CompilerParams(needs_layout_passes=False),
    )
    return fn(ss_flat, group_id.astype(jnp.int32), config_idx.astype(jnp.int32),
              ipt.astype(jnp.float32), z, attn_w)
